# Initial kernel scaffold; baseline (speedup 1.0000x reference)
import functools

import jax
import jax.numpy as jnp
import numpy as np
from jax import lax
from jax.experimental import pallas as pl
from jax.experimental.pallas import tpu as pltpu
from jax.experimental.pallas import tpu_sc as plsc

_DIM = 32
_SCALE = float(np.sqrt(_DIM))

_NC, _NS = 2, 16
_NW = _NC * _NS

_ROWS_PER_STREAM = 128
_STREAMS = 16
_CHUNK = _ROWS_PER_STREAM * _STREAMS


def _sc_gather_scale(x_flat, table):
    n_total = x_flat.shape[0]
    per_w = n_total // _NW
    n_chunks = per_w // _CHUNK
    assert per_w % _CHUNK == 0

    mesh = plsc.VectorSubcoreMesh(
        core_axis_name="c", subcore_axis_name="s",
        num_cores=_NC, num_subcores=_NS,
    )

    @functools.partial(
        pl.kernel,
        out_type=jax.ShapeDtypeStruct((n_total, _DIM), jnp.float32),
        mesh=mesh,
        scratch_types=[
            pltpu.VMEM((_STREAMS, _ROWS_PER_STREAM), jnp.int32),
            pltpu.VMEM((_CHUNK, _DIM), jnp.float32),
            pltpu.SemaphoreType.DMA,
        ],
    )
    def k(x_hbm, table_hbm, out_hbm, idx_v, rows_v, sem):
        wid = lax.axis_index("s") * _NC + lax.axis_index("c")
        base = wid * per_w

        def chunk_body(i, carry):
            off = base + i * _CHUNK
            pltpu.sync_copy(x_hbm.at[pl.ds(off, _CHUNK)], idx_v)
            copies = []
            for j in range(_STREAMS):
                copies.append(pltpu.async_copy(
                    table_hbm.at[idx_v.at[j]],
                    rows_v.at[pl.ds(j * _ROWS_PER_STREAM, _ROWS_PER_STREAM)],
                    sem,
                ))
            for c in copies:
                c.wait()

            def scale_body(r, carry2):
                rows_v[r, pl.ds(0, 16)] = rows_v[r, pl.ds(0, 16)] * _SCALE
                rows_v[r, pl.ds(16, 16)] = rows_v[r, pl.ds(16, 16)] * _SCALE
                return carry2

            lax.fori_loop(0, _CHUNK, scale_body, 0, unroll=4)
            pltpu.sync_copy(rows_v, out_hbm.at[pl.ds(off, _CHUNK)])
            return carry

        lax.fori_loop(0, n_chunks, chunk_body, 0)

    return k(x_flat, table)


def kernel(x, table):
    x_flat = x.reshape(-1)
    out = _sc_gather_scale(x_flat, table)
    return out.reshape(x.shape + (_DIM,))


# speedup vs baseline: 4.6938x; 4.6938x over previous
import functools

import jax
import jax.numpy as jnp
import numpy as np
from jax import lax
from jax.experimental import pallas as pl
from jax.experimental.pallas import tpu as pltpu
from jax.experimental.pallas import tpu_sc as plsc

_DIM = 32
_SCALE = float(np.sqrt(_DIM))

_NC, _NS = 2, 16
_NW = _NC * _NS

_ROWS_PER_STREAM = 128
_STREAMS = 16
_CHUNK = _ROWS_PER_STREAM * _STREAMS


def _sc_gather_scale(x_flat, table):
    n_total = x_flat.shape[0]
    per_w = n_total // _NW
    n_chunks = per_w // _CHUNK
    assert per_w % _CHUNK == 0

    mesh = plsc.VectorSubcoreMesh(
        core_axis_name="c", subcore_axis_name="s",
        num_cores=_NC, num_subcores=_NS,
    )

    @functools.partial(
        pl.kernel,
        out_type=jax.ShapeDtypeStruct((n_total, _DIM), jnp.float32),
        mesh=mesh,
        scratch_types=[
            pltpu.VMEM((_STREAMS, _ROWS_PER_STREAM), jnp.int32),
            pltpu.VMEM((_CHUNK, _DIM), jnp.float32),
            pltpu.SemaphoreType.DMA,
        ],
        compiler_params=pltpu.CompilerParams(use_tc_tiling_on_sc=False),
    )
    def k(x_hbm, table_hbm, out_hbm, idx_v, rows_v, sem):
        wid = lax.axis_index("s") * _NC + lax.axis_index("c")
        base = wid * per_w

        def chunk_body(i, carry):
            off = base + i * _CHUNK
            row_off = pl.multiple_of(off // _ROWS_PER_STREAM, _STREAMS)
            pltpu.sync_copy(x_hbm.at[pl.ds(row_off, _STREAMS)], idx_v)
            copies = []
            for j in range(_STREAMS):
                copies.append(pltpu.async_copy(
                    table_hbm.at[idx_v.at[j]],
                    rows_v.at[pl.ds(j * _ROWS_PER_STREAM, _ROWS_PER_STREAM)],
                    sem,
                ))
            for c in copies:
                c.wait()

            def scale_body(r, carry2):
                rows_v[r, pl.ds(0, 16)] = rows_v[r, pl.ds(0, 16)] * _SCALE
                rows_v[r, pl.ds(16, 16)] = rows_v[r, pl.ds(16, 16)] * _SCALE
                return carry2

            lax.fori_loop(0, _CHUNK, scale_body, 0, unroll=4)
            pltpu.sync_copy(rows_v, out_hbm.at[pl.ds(off, _CHUNK)])
            return carry

        lax.fori_loop(0, n_chunks, chunk_body, 0)

    return k(x_flat.reshape(-1, _ROWS_PER_STREAM), table)


def kernel(x, table):
    x_flat = x.reshape(-1)
    out = _sc_gather_scale(x_flat, table)
    return out.reshape(x.shape + (_DIM,))


# speedup vs baseline: 5.0082x; 1.0670x over previous
import functools

import jax
import jax.numpy as jnp
import numpy as np
from jax import lax
from jax.experimental import pallas as pl
from jax.experimental.pallas import tpu as pltpu
from jax.experimental.pallas import tpu_sc as plsc

_DIM = 32
_SCALE = float(np.sqrt(_DIM))

_NC, _NS = 2, 16
_NW = _NC * _NS

_ROWS_PER_STREAM = 128
_STREAMS = 10
_CHUNK = _ROWS_PER_STREAM * _STREAMS


def _sc_gather_scale(x_flat, table):
    n_total = x_flat.shape[0]
    per_w = n_total // _NW
    n_chunks = per_w // _CHUNK
    assert per_w % _CHUNK == 0 and n_chunks % 2 == 0

    mesh = plsc.VectorSubcoreMesh(
        core_axis_name="c", subcore_axis_name="s",
        num_cores=_NC, num_subcores=_NS,
    )

    @functools.partial(
        pl.kernel,
        out_type=jax.ShapeDtypeStruct((n_total, _DIM), jnp.float32),
        mesh=mesh,
        scratch_types=[
            pltpu.VMEM((_STREAMS, _ROWS_PER_STREAM), jnp.int32),
            pltpu.VMEM((_STREAMS, _ROWS_PER_STREAM), jnp.int32),
            pltpu.VMEM((_CHUNK, _DIM), jnp.float32),
            pltpu.VMEM((_CHUNK, _DIM), jnp.float32),
            pltpu.SemaphoreType.DMA,
            pltpu.SemaphoreType.DMA,
            pltpu.SemaphoreType.DMA,
            pltpu.SemaphoreType.DMA,
        ],
        compiler_params=pltpu.CompilerParams(use_tc_tiling_on_sc=False),
    )
    def k(x_hbm, table_hbm, out_hbm, idx0, idx1, rows0, rows1,
          gsem0, gsem1, osem0, osem1):
        wid = lax.axis_index("s") * _NC + lax.axis_index("c")
        base = wid * per_w
        idx_bufs = (idx0, idx1)
        row_bufs = (rows0, rows1)
        gsems = (gsem0, gsem1)
        osems = (osem0, osem1)

        def load_idx(c, buf):
            row_off = (base + c * _CHUNK) // _ROWS_PER_STREAM
            pltpu.sync_copy(x_hbm.at[pl.ds(row_off, _STREAMS)], idx_bufs[buf])

        def fire_gathers(buf):
            for j in range(_STREAMS):
                pltpu.async_copy(
                    table_hbm.at[idx_bufs[buf].at[j]],
                    row_bufs[buf].at[pl.ds(j * _ROWS_PER_STREAM,
                                           _ROWS_PER_STREAM)],
                    gsems[buf],
                )

        def drain_gathers(buf):
            pltpu.make_async_copy(
                out_hbm.at[pl.ds(0, _CHUNK)], row_bufs[buf], gsems[buf],
            ).wait()

        def fire_scatter(c, buf):
            off = base + c * _CHUNK
            pltpu.async_copy(
                row_bufs[buf], out_hbm.at[pl.ds(off, _CHUNK)], osems[buf])

        def drain_scatter(buf):
            pltpu.make_async_copy(
                row_bufs[buf], out_hbm.at[pl.ds(0, _CHUNK)], osems[buf],
            ).wait()

        def scale(buf):
            rows = row_bufs[buf]

            def scale_body(r, carry):
                rows[r, pl.ds(0, 16)] = rows[r, pl.ds(0, 16)] * _SCALE
                rows[r, pl.ds(16, 16)] = rows[r, pl.ds(16, 16)] * _SCALE
                return carry

            lax.fori_loop(0, _CHUNK, scale_body, 0, unroll=4)

        load_idx(0, 0)
        fire_gathers(0)

        def pair_body(g, carry):
            for b in (0, 1):
                c = g * 2 + b
                nb = 1 - b
                @pl.when(c + 1 < n_chunks)
                def _():
                    load_idx(c + 1, nb)
                @pl.when(c >= 1)
                def _():
                    drain_scatter(nb)
                @pl.when(c + 1 < n_chunks)
                def _():
                    fire_gathers(nb)
                drain_gathers(b)
                scale(b)
                fire_scatter(c, b)
            return carry

        lax.fori_loop(0, n_chunks // 2, pair_body, 0)
        drain_scatter(1)

    return k(x_flat.reshape(-1, _ROWS_PER_STREAM), table)


def kernel(x, table):
    x_flat = x.reshape(-1)
    out = _sc_gather_scale(x_flat, table)
    return out.reshape(x.shape + (_DIM,))
